# baseline (speedup 1.0000x reference)
import functools

import jax
import jax.numpy as jnp
from jax.experimental import pallas as pl
from jax.experimental.pallas import tpu as pltpu

_VMEM_LIMIT_BYTES = 64 * 1024 * 1024
_NCORES = 2


def _layer1_kernel(adj_ref, b_ref, w_ref, adjb_ref, out_ref, bb_ref, wb_ref):
    @pl.when(pl.program_id(1) == 0)
    def _():
        bb_ref[...] = b_ref[...].astype(jnp.bfloat16)
        wb_ref[...] = w_ref[...].astype(jnp.bfloat16)

    adj_b = adj_ref[...].astype(jnp.bfloat16)
    adjb_ref[...] = adj_b
    h = jnp.dot(adj_b, bb_ref[...], preferred_element_type=jnp.float32)
    h = jnp.maximum(h, 0.0).astype(jnp.bfloat16)
    out_ref[...] = jnp.dot(
        h, wb_ref[...], preferred_element_type=jnp.float32
    ).astype(out_ref.dtype)


def _layer23_kernel(adjr_ref, b_ref, w_ref, adjc_ref, out_ref):
    h = jnp.dot(adjr_ref[...], b_ref[...], preferred_element_type=jnp.float32)
    h = jnp.maximum(h, 0.0).astype(jnp.bfloat16)
    pre2 = jnp.dot(
        h, w_ref[...].astype(jnp.bfloat16), preferred_element_type=jnp.float32
    ).astype(jnp.bfloat16)
    out_ref[...] = jnp.dot(
        adjc_ref[...], pre2, preferred_element_type=jnp.float32
    )[None]


def _grid_specs(m, tm):
    s = m // tm // _NCORES
    grid = (_NCORES, s)
    row = pl.BlockSpec((tm, None), lambda i, j, _s=s: (i * _s + j, 0))
    return grid, row, s


def _layer1(adj_f32, b, w, *, tm):
    m, k = adj_f32.shape
    h = b.shape[1]
    c = w.shape[1]
    s = m // tm // _NCORES
    idx = lambda i, j: (i * s + j, 0)
    const = lambda i, j: (0, 0)
    return pl.pallas_call(
        _layer1_kernel,
        out_shape=(
            jax.ShapeDtypeStruct((m, k), jnp.bfloat16),
            jax.ShapeDtypeStruct((m, c), jnp.bfloat16),
        ),
        grid=(_NCORES, s),
        in_specs=[
            pl.BlockSpec((tm, k), idx),
            pl.BlockSpec((k, h), const),
            pl.BlockSpec((h, c), const),
        ],
        out_specs=(
            pl.BlockSpec((tm, k), idx),
            pl.BlockSpec((tm, c), idx),
        ),
        scratch_shapes=[
            pltpu.VMEM((k, h), jnp.bfloat16),
            pltpu.VMEM((h, c), jnp.bfloat16),
        ],
        compiler_params=pltpu.CompilerParams(
            dimension_semantics=("parallel", "arbitrary"),
            vmem_limit_bytes=_VMEM_LIMIT_BYTES,
        ),
    )(adj_f32, b, w)


def _layer23(adj_b, b, w):
    m, k = adj_b.shape
    h = b.shape[1]
    c = w.shape[1]
    half = m // _NCORES
    const = lambda i: (0, 0)
    return pl.pallas_call(
        _layer23_kernel,
        out_shape=jax.ShapeDtypeStruct((_NCORES, m, c), jnp.float32),
        grid=(_NCORES,),
        in_specs=[
            pl.BlockSpec((half, k), lambda i: (i, 0)),
            pl.BlockSpec((k, h), const),
            pl.BlockSpec((h, c), const),
            pl.BlockSpec((k, half), lambda i: (0, i)),
        ],
        out_specs=pl.BlockSpec((1, m, c), lambda i: (i, 0, 0)),
        compiler_params=pltpu.CompilerParams(
            dimension_semantics=("parallel",),
            vmem_limit_bytes=_VMEM_LIMIT_BYTES,
        ),
    )(adj_b, b, w, adj_b)


def kernel(W1, W2, Wmid0, x, adj):
    del x
    n = adj.shape[0]
    assert n % 512 == 0, adj.shape
    tm = n // _NCORES

    adj_b, pre1 = _layer1(adj, W1, Wmid0, tm=tm)
    parts = _layer23(adj_b, pre1, W2)
    return parts[0] + parts[1]


# speedup vs baseline: 1.3831x; 1.3831x over previous
import functools

import jax
import jax.numpy as jnp
from jax.experimental import pallas as pl
from jax.experimental.pallas import tpu as pltpu

_VMEM_LIMIT_BYTES = 100 * 1024 * 1024


def _gcn_kernel(adj_ref, w1_ref, wm_ref, w2_ref, out_ref,
                adjb_ref, w1b_ref, wmb_ref, w2b_ref, pre1_ref, pre2_ref,
                *, tm):
    p = pl.program_id(0)
    j = pl.program_id(1)
    rows = pl.ds(j * tm, tm)

    @pl.when((p == 0) & (j == 0))
    def _():
        w1b_ref[...] = w1_ref[...].astype(jnp.bfloat16)
        wmb_ref[...] = wm_ref[...].astype(jnp.bfloat16)
        w2b_ref[...] = w2_ref[...].astype(jnp.bfloat16)

    @pl.when(p == 0)
    def _():
        a = adj_ref[...].astype(jnp.bfloat16)
        adjb_ref[rows, :] = a
        h = jnp.dot(a, w1b_ref[...], preferred_element_type=jnp.float32)
        h = jnp.maximum(h, 0.0).astype(jnp.bfloat16)
        pre1_ref[rows, :] = jnp.dot(
            h, wmb_ref[...], preferred_element_type=jnp.float32
        ).astype(jnp.bfloat16)

    @pl.when(p == 1)
    def _():
        h = jnp.dot(adjb_ref[rows, :], pre1_ref[...],
                    preferred_element_type=jnp.float32)
        h = jnp.maximum(h, 0.0).astype(jnp.bfloat16)
        pre2_ref[rows, :] = jnp.dot(
            h, w2b_ref[...], preferred_element_type=jnp.float32
        ).astype(jnp.bfloat16)

    @pl.when(p == 2)
    def _():
        out_ref[...] = jnp.dot(adjb_ref[rows, :], pre2_ref[...],
                               preferred_element_type=jnp.float32)


def kernel(W1, W2, Wmid0, x, adj):
    del x
    n, k = adj.shape
    h = W1.shape[1]
    c = W2.shape[1]
    tm = min(512, n)
    s = n // tm
    assert n % tm == 0, adj.shape

    adj_idx = lambda p, j: (jnp.where(p == 0, j, s - 1), 0)
    const = lambda p, j: (0, 0)

    return pl.pallas_call(
        functools.partial(_gcn_kernel, tm=tm),
        out_shape=jax.ShapeDtypeStruct((n, c), jnp.float32),
        grid=(3, s),
        in_specs=[
            pl.BlockSpec((tm, k), adj_idx),
            pl.BlockSpec((k, h), const),
            pl.BlockSpec((h, h), const),
            pl.BlockSpec((h, c), const),
        ],
        out_specs=pl.BlockSpec((tm, c), lambda p, j: (j, 0)),
        scratch_shapes=[
            pltpu.VMEM((n, k), jnp.bfloat16),
            pltpu.VMEM((k, h), jnp.bfloat16),
            pltpu.VMEM((h, h), jnp.bfloat16),
            pltpu.VMEM((h, c), jnp.bfloat16),
            pltpu.VMEM((n, h), jnp.bfloat16),
            pltpu.VMEM((n, c), jnp.bfloat16),
        ],
        compiler_params=pltpu.CompilerParams(
            dimension_semantics=("arbitrary", "arbitrary"),
            vmem_limit_bytes=_VMEM_LIMIT_BYTES,
        ),
    )(adj, W1, Wmid0, W2)
